# baseline (speedup 1.0000x reference)
import functools

import jax
import jax.numpy as jnp
from jax import lax
from jax.experimental import pallas as pl
from jax.experimental.pallas import tpu as pltpu
from jax.experimental.pallas import tpu_sc as plsc

NC = 2
NS = 16
NW = NC * NS
LANES = 16
GCHUNK = 128
NLQ = 8
NBQ = 4


def kernel(sequence, segment_labels, tok_table, seg_table, pe):
    B, L = sequence.shape
    V, D = tok_table.shape
    N = B * L
    n_seg = seg_table.shape[0]
    n_jb = D // LANES
    lpw = L // NLQ
    bpw = B // NBQ
    n_bu = bpw // LANES
    n_td = D // 8
    ntb = B // 128
    assert L % NLQ == 0 and B % NBQ == 0 and bpw % GCHUNK == 0 and D % 8 == 0
    assert lpw >= 4 and lpw % 2 == 1

    seqT = jnp.transpose(sequence).astype(jnp.int32)
    lblT = jnp.transpose(segment_labels).astype(jnp.int32)
    pe_flat = pe.reshape(pe.shape[1], D)[:L].reshape(L * D)
    seg_flat = seg_table.astype(jnp.float32).reshape(n_seg * D)

    mesh = plsc.VectorSubcoreMesh(core_axis_name="c", subcore_axis_name="s")

    @functools.partial(
        pl.kernel,
        out_type=jax.ShapeDtypeStruct((N * D,), jnp.float32),
        mesh=mesh,
        compiler_params=pltpu.CompilerParams(
            needs_layout_passes=False, use_tc_tiling_on_sc=False),
        scratch_types=[
            pltpu.VMEM((lpw, bpw), jnp.int32),
            pltpu.VMEM((lpw, bpw), jnp.int32),
            pltpu.VMEM((lpw * D,), jnp.float32),
            pltpu.VMEM((n_seg * D,), jnp.float32),
            pltpu.VMEM((n_seg * D,), jnp.float32),
            pltpu.VMEM((bpw, D), jnp.float32),
            pltpu.VMEM((bpw, D), jnp.float32),
            pltpu.VMEM((bpw * (D + 1),), jnp.float32),
            pltpu.VMEM((n_td, 8 * 256), jnp.float32),
            pltpu.VMEM((n_td, 8 * 256), jnp.float32),
            pltpu.SemaphoreType.DMA,
            pltpu.SemaphoreType.DMA,
            pltpu.SemaphoreType.DMA,
            pltpu.SemaphoreType.DMA,
            pltpu.SemaphoreType.DMA,
        ],
    )
    def run(seq_ref, lbl_ref, tok_ref, seg_ref, pe_ref, out_ref,
            idx_v, lbl_v, pe_v, seg_v, combo_v, rows0, rows1, rows_p, ob0, ob1,
            sem_in, sem_g0, sem_g1, sem_o0, sem_o1):
        wid = lax.axis_index("s") * NC + lax.axis_index("c")
        lq = lax.rem(wid, NLQ)
        bq = wid // NLQ
        l0 = lq * lpw
        b0 = bq * bpw
        iota = lax.iota(jnp.int32, LANES)

        cp_i = pltpu.async_copy(
            seq_ref.at[pl.ds(l0, lpw), pl.ds(b0, bpw)], idx_v, sem_in)
        cp_l = pltpu.async_copy(
            lbl_ref.at[pl.ds(l0, lpw), pl.ds(b0, bpw)], lbl_v, sem_in)
        pltpu.sync_copy(pe_ref.at[pl.ds(l0 * D, lpw * D)], pe_v)
        pltpu.sync_copy(seg_ref, seg_v)
        cp_i.wait()
        cp_l.wait()

        rows = (rows0, rows1)
        obufs = (ob0, ob1)
        sem_g = (sem_g0, sem_g1)
        sem_o = (sem_o0, sem_o1)
        out_col = bq * (8 * bpw)

        def g_issue(l_off, p):
            for k in range(bpw // GCHUNK):
                pltpu.async_copy(
                    tok_ref.at[idx_v.at[l_off, pl.ds(k * GCHUNK, GCHUNK)]],
                    rows[p].at[pl.ds(k * GCHUNK, GCHUNK), :], sem_g[p])

        def g_wait(l_off, p):
            for k in range(bpw // GCHUNK):
                pltpu.make_async_copy(
                    tok_ref.at[idx_v.at[l_off, pl.ds(k * GCHUNK, GCHUNK)]],
                    rows[p].at[pl.ds(k * GCHUNK, GCHUNK), :], sem_g[p]).wait()

        def o_issue(l_off, p):
            base = (l0 + l_off) * (D * B) + out_col
            for td in range(n_td):
                pltpu.async_copy(
                    obufs[p].at[td],
                    out_ref.at[pl.ds(base + td * (8 * B), 8 * bpw)], sem_o[p])

        def o_wait(l_off, p):
            base = (l0 + l_off) * (D * B) + out_col
            for td in range(n_td):
                pltpu.make_async_copy(
                    obufs[p].at[td],
                    out_ref.at[pl.ds(base + td * (8 * B), 8 * bpw)],
                    sem_o[p]).wait()

        bu65 = [(iota + u * LANES) * (D + 1) for u in range(n_bu)]

        def compute_block(l_off, p):
            rows_b = rows[p]
            obuf = obufs[p]

            def repack_body(b4, c3):
                for v in range(4):
                    b = b4 * 4 + v
                    for jb in range(n_jb):
                        rows_p[pl.ds(b * (D + 1) + jb * LANES, LANES)] = (
                            rows_b[b, pl.ds(jb * LANES, LANES)])
                return c3

            lax.fori_loop(0, bpw // 4, repack_body, 0)
            for s in range(n_seg):
                for jb in range(n_jb):
                    pv = pe_v[pl.ds(l_off * D + jb * LANES, LANES)]
                    sv = seg_v[pl.ds(s * D + jb * LANES, LANES)]
                    combo_v[pl.ds(s * D + jb * LANES, LANES)] = pv + sv
            cus = [lbl_v[l_off, pl.ds(u * LANES, LANES)] * D
                   for u in range(n_bu)]

            def td_body(td, c2):
                for ds in range(8):
                    dsplat = jnp.full((LANES,), td * 8 + ds, dtype=jnp.int32)
                    for u in range(n_bu):
                        tok = plsc.load_gather(rows_p, [bu65[u] + dsplat])
                        add = plsc.load_gather(combo_v, [cus[u] + dsplat])
                        off = (u // 8) * 1024 + ds * 128 + (u % 8) * LANES
                        obuf[td, pl.ds(off, LANES)] = tok + add
                return c2

            lax.fori_loop(0, n_td, td_body, 0)

        def steady(i, p):
            q = 1 - p
            o_wait(i - 1, q)
            g_issue(i + 1, q)
            g_wait(i, p)
            compute_block(i, p)
            o_issue(i, p)

        g_issue(0, 0)
        g_issue(1, 1)
        g_wait(0, 0)
        compute_block(0, 0)
        o_issue(0, 0)

        def block(k, carry):
            steady(1 + 2 * k, 1)
            steady(2 + 2 * k, 0)
            return carry

        lax.fori_loop(0, (lpw - 3) // 2, block, 0)

        steady(lpw - 2, 1)

        o_wait(lpw - 2, 1)
        g_wait(lpw - 1, 0)
        compute_block(lpw - 1, 0)
        o_issue(lpw - 1, 0)
        o_wait(lpw - 1, 0)

    out_flat = run(seqT, lblT, tok_table, seg_flat, pe_flat)
    t5 = out_flat.reshape(L, n_td, ntb, 8, 128)
    return jnp.transpose(t5, (2, 4, 0, 1, 3)).reshape(B, L, D)


# speedup vs baseline: 1.2542x; 1.2542x over previous
import functools

import jax
import jax.numpy as jnp
from jax import lax
from jax.experimental import pallas as pl
from jax.experimental.pallas import tpu as pltpu
from jax.experimental.pallas import tpu_sc as plsc

NC = 2
NS = 16
NW = NC * NS
LANES = 16
GCHUNK = 128
CHUNK = 256
NG_PER = CHUNK // GCHUNK


def kernel(sequence, segment_labels, tok_table, seg_table, pe):
    B, L = sequence.shape
    V, D = tok_table.shape
    N = B * L
    n_groups = CHUNK // LANES
    rows_per_w = N // NW
    n_chunks = rows_per_w // CHUNK
    n_seg = seg_table.shape[0]
    n_jb = D // LANES
    assert N == NW * n_chunks * CHUNK and D % LANES == 0
    assert n_chunks >= 5 and n_chunks % 3 == 1

    seq_flat = sequence.reshape(N).astype(jnp.int32)
    lbl_flat = segment_labels.reshape(N).astype(jnp.int32)
    pe_flat = pe.reshape(pe.shape[1], D)[:L].reshape(L * D)
    seg_flat = seg_table.astype(jnp.float32).reshape(n_seg * D)

    mesh = plsc.VectorSubcoreMesh(core_axis_name="c", subcore_axis_name="s")

    @functools.partial(
        pl.kernel,
        out_type=jax.ShapeDtypeStruct((N, D), jnp.float32),
        mesh=mesh,
        compiler_params=pltpu.CompilerParams(
            needs_layout_passes=False, use_tc_tiling_on_sc=False),
        scratch_types=[
            pltpu.VMEM((n_seg * L * D,), jnp.float32),
            pltpu.VMEM((n_seg * D,), jnp.float32),
            pltpu.VMEM((rows_per_w,), jnp.int32),
            pltpu.VMEM((rows_per_w,), jnp.int32),
            pltpu.VMEM((CHUNK, D), jnp.float32),
            pltpu.VMEM((CHUNK, D), jnp.float32),
            pltpu.VMEM((CHUNK, D), jnp.float32),
            pltpu.SemaphoreType.DMA,
            pltpu.SemaphoreType.DMA,
            pltpu.SemaphoreType.DMA,
            pltpu.SemaphoreType.DMA,
            pltpu.SemaphoreType.DMA,
            pltpu.SemaphoreType.DMA,
            pltpu.SemaphoreType.DMA,
        ],
    )
    def run(seq_ref, lbl_ref, tok_ref, seg_ref, pe_ref, out_ref,
            combo_v, seg_v, idx_all, lbl_all, rows0, rows1, rows2,
            sem_in, sem_g0, sem_g1, sem_g2, sem_o0, sem_o1, sem_o2):
        wid = lax.axis_index("s") * NC + lax.axis_index("c")
        iota = lax.iota(jnp.int32, LANES)
        wbase = wid * rows_per_w

        cp_i = pltpu.async_copy(
            seq_ref.at[pl.ds(wbase, rows_per_w)], idx_all, sem_in)
        cp_l = pltpu.async_copy(
            lbl_ref.at[pl.ds(wbase, rows_per_w)], lbl_all, sem_in)

        pltpu.sync_copy(seg_ref, seg_v)
        for s in range(n_seg):
            pltpu.sync_copy(pe_ref, combo_v.at[pl.ds(s * L * D, L * D)])
        seg_rows = [[seg_v[pl.ds(s * D + jb * LANES, LANES)]
                     for jb in range(n_jb)] for s in range(n_seg)]

        def build_body(l, carry):
            for s in range(n_seg):
                for jb in range(n_jb):
                    sl = pl.ds(s * L * D + l * D + jb * LANES, LANES)
                    combo_v[sl] = combo_v[sl] + seg_rows[s][jb]
            return carry

        lax.fori_loop(0, L, build_body, 0)
        cp_i.wait()
        cp_l.wait()

        rows = (rows0, rows1, rows2)
        sem_g = (sem_g0, sem_g1, sem_g2)
        sem_o = (sem_o0, sem_o1, sem_o2)
        col_iotas = [iota + jb * LANES for jb in range(n_jb)]

        def g_issue(loc, p):
            for k in range(NG_PER):
                pltpu.async_copy(
                    tok_ref.at[idx_all.at[pl.ds(loc + k * GCHUNK, GCHUNK)]],
                    rows[p].at[pl.ds(k * GCHUNK, GCHUNK), :], sem_g[p])

        def g_wait(loc, p):
            for k in range(NG_PER):
                pltpu.make_async_copy(
                    tok_ref.at[idx_all.at[pl.ds(loc + k * GCHUNK, GCHUNK)]],
                    rows[p].at[pl.ds(k * GCHUNK, GCHUNK), :], sem_g[p]).wait()

        def o_issue(loc, p):
            pltpu.async_copy(
                rows[p], out_ref.at[pl.ds(wbase + loc, CHUNK), :], sem_o[p])

        def o_wait(loc, p):
            pltpu.make_async_copy(
                rows[p], out_ref.at[pl.ds(wbase + loc, CHUNK), :],
                sem_o[p]).wait()

        def compute_chunk(loc, rows_buf):
            gbase = wbase + loc

            def group_body(g, c2):
                lvec = lax.rem(gbase + g * LANES + iota, L)
                svec = lbl_all[pl.ds(loc + g * LANES, LANES)]
                cvals = (svec * L + lvec) * D
                for r in range(LANES):
                    row = g * LANES + r
                    cbase = cvals.at[jnp.full((LANES,), r, dtype=jnp.int32)].get(
                        mode="promise_in_bounds")
                    for jb in range(n_jb):
                        add = plsc.load_gather(combo_v, [cbase + col_iotas[jb]])
                        plsc.addupdate(
                            rows_buf.at[row, pl.ds(jb * LANES, LANES)], add)
                return c2

            lax.fori_loop(0, n_groups, group_body, 0)

        def steady(i, p):
            q2 = (p + 2) % 3
            loc = i * CHUNK
            o_wait(loc - CHUNK, q2)
            g_issue(loc + 2 * CHUNK, q2)
            g_wait(loc, p)
            compute_chunk(loc, rows[p])
            o_issue(loc, p)

        g_issue(0, 0)
        g_issue(CHUNK, 1)
        g_issue(2 * CHUNK, 2)
        g_wait(0, 0)
        compute_chunk(0, rows0)
        o_issue(0, 0)
        o_wait(0, 0)
        g_issue(3 * CHUNK, 0)
        g_wait(CHUNK, 1)
        compute_chunk(CHUNK, rows1)
        o_issue(CHUNK, 1)

        def block(k, carry):
            steady(2 + 3 * k, 2)
            steady(3 + 3 * k, 0)
            steady(4 + 3 * k, 1)
            return carry

        lax.fori_loop(0, (n_chunks - 4) // 3, block, 0)

        loc = (n_chunks - 2) * CHUNK
        o_wait(loc - CHUNK, 1)
        g_wait(loc, 2)
        compute_chunk(loc, rows2)
        o_issue(loc, 2)

        loc = (n_chunks - 1) * CHUNK
        o_wait(loc - CHUNK, 2)
        g_wait(loc, 0)
        compute_chunk(loc, rows0)
        o_issue(loc, 0)
        o_wait(loc, 0)

    out = run(seq_flat, lbl_flat, tok_table, seg_flat, pe_flat)
    return out.reshape(B, L, D)


# speedup vs baseline: 1.2546x; 1.0003x over previous
import functools

import jax
import jax.numpy as jnp
from jax import lax
from jax.experimental import pallas as pl
from jax.experimental.pallas import tpu as pltpu
from jax.experimental.pallas import tpu_sc as plsc

NC = 2
NS = 16
NW = NC * NS
LANES = 16
GCHUNK = 128
CHUNK = 256
NG_PER = CHUNK // GCHUNK


def kernel(sequence, segment_labels, tok_table, seg_table, pe):
    B, L = sequence.shape
    V, D = tok_table.shape
    N = B * L
    n_groups = CHUNK // LANES
    rows_per_w = N // NW
    n_chunks = rows_per_w // CHUNK
    n_seg = seg_table.shape[0]
    n_jb = D // LANES
    assert N == NW * n_chunks * CHUNK and D % LANES == 0
    assert n_chunks >= 4 and n_chunks % 2 == 1

    seq_flat = sequence.reshape(N).astype(jnp.int32)
    lbl_flat = segment_labels.reshape(N).astype(jnp.int32)
    pe_flat = pe.reshape(pe.shape[1], D)[:L].reshape(L * D)
    seg_flat = seg_table.astype(jnp.float32).reshape(n_seg * D)

    mesh = plsc.VectorSubcoreMesh(core_axis_name="c", subcore_axis_name="s")

    @functools.partial(
        pl.kernel,
        out_type=jax.ShapeDtypeStruct((N, D), jnp.float32),
        mesh=mesh,
        compiler_params=pltpu.CompilerParams(
            needs_layout_passes=False, use_tc_tiling_on_sc=False),
        scratch_types=[
            pltpu.VMEM((n_seg * L * D,), jnp.float32),
            pltpu.VMEM((n_seg * D,), jnp.float32),
            pltpu.VMEM((rows_per_w,), jnp.int32),
            pltpu.VMEM((rows_per_w,), jnp.int32),
            pltpu.VMEM((CHUNK, D), jnp.float32),
            pltpu.VMEM((CHUNK, D), jnp.float32),
            pltpu.SemaphoreType.DMA,
            pltpu.SemaphoreType.DMA,
            pltpu.SemaphoreType.DMA,
            pltpu.SemaphoreType.DMA,
            pltpu.SemaphoreType.DMA,
        ],
    )
    def run(seq_ref, lbl_ref, tok_ref, seg_ref, pe_ref, out_ref,
            combo_v, seg_v, idx_all, lbl_all, rows0, rows1,
            sem_in, sem_g0, sem_g1, sem_o0, sem_o1):
        wid = lax.axis_index("s") * NC + lax.axis_index("c")
        iota = lax.iota(jnp.int32, LANES)
        wbase = wid * rows_per_w

        cp_i = pltpu.async_copy(
            seq_ref.at[pl.ds(wbase, rows_per_w)], idx_all, sem_in)
        cp_l = pltpu.async_copy(
            lbl_ref.at[pl.ds(wbase, rows_per_w)], lbl_all, sem_in)

        pltpu.sync_copy(seg_ref, seg_v)
        for s in range(n_seg):
            pltpu.sync_copy(pe_ref, combo_v.at[pl.ds(s * L * D, L * D)])
        seg_rows = [[seg_v[pl.ds(s * D + jb * LANES, LANES)]
                     for jb in range(n_jb)] for s in range(n_seg)]

        def build_body(l, carry):
            for s in range(n_seg):
                for jb in range(n_jb):
                    sl = pl.ds(s * L * D + l * D + jb * LANES, LANES)
                    combo_v[sl] = combo_v[sl] + seg_rows[s][jb]
            return carry

        lax.fori_loop(0, L, build_body, 0)
        cp_i.wait()
        cp_l.wait()

        rows = (rows0, rows1)
        sem_g = (sem_g0, sem_g1)
        sem_o = (sem_o0, sem_o1)
        col_iotas = [iota + jb * LANES for jb in range(n_jb)]

        def g_issue(loc, p):
            for k in range(NG_PER):
                pltpu.async_copy(
                    tok_ref.at[idx_all.at[pl.ds(loc + k * GCHUNK, GCHUNK)]],
                    rows[p].at[pl.ds(k * GCHUNK, GCHUNK), :], sem_g[p])

        def g_wait(loc, p):
            for k in range(NG_PER):
                pltpu.make_async_copy(
                    tok_ref.at[idx_all.at[pl.ds(loc + k * GCHUNK, GCHUNK)]],
                    rows[p].at[pl.ds(k * GCHUNK, GCHUNK), :], sem_g[p]).wait()

        def o_issue(loc, p):
            pltpu.async_copy(
                rows[p], out_ref.at[pl.ds(wbase + loc, CHUNK), :], sem_o[p])

        def o_wait(loc, p):
            pltpu.make_async_copy(
                rows[p], out_ref.at[pl.ds(wbase + loc, CHUNK), :],
                sem_o[p]).wait()

        def compute_chunk(loc, rows_buf):
            gbase = wbase + loc

            def group_body(g, c2):
                lvec = lax.rem(gbase + g * LANES + iota, L)
                svec = lbl_all[pl.ds(loc + g * LANES, LANES)]
                cvals = (svec * L + lvec) * D
                for r in range(LANES):
                    row = g * LANES + r
                    cbase = cvals.at[jnp.full((LANES,), r, dtype=jnp.int32)].get(
                        mode="promise_in_bounds")
                    for jb in range(n_jb):
                        add = plsc.load_gather(combo_v, [cbase + col_iotas[jb]])
                        plsc.addupdate(
                            rows_buf.at[row, pl.ds(jb * LANES, LANES)], add)
                return c2

            lax.fori_loop(0, n_groups, group_body, 0)

        def steady(i, p):
            q = 1 - p
            loc = i * CHUNK
            o_wait(loc - CHUNK, q)
            g_issue(loc + CHUNK, q)
            g_wait(loc, p)
            compute_chunk(loc, rows[p])
            o_issue(loc, p)

        g_issue(0, 0)
        g_issue(CHUNK, 1)
        g_wait(0, 0)
        compute_chunk(0, rows0)
        o_issue(0, 0)

        def block(k, carry):
            steady(1 + 2 * k, 1)
            steady(2 + 2 * k, 0)
            return carry

        lax.fori_loop(0, (n_chunks - 3) // 2, block, 0)

        steady(n_chunks - 2, 1)

        loc = (n_chunks - 1) * CHUNK
        o_wait(loc - CHUNK, 1)
        g_wait(loc, 0)
        compute_chunk(loc, rows0)
        o_issue(loc, 0)
        o_wait(loc, 0)

    out = run(seq_flat, lbl_flat, tok_table, seg_flat, pe_flat)
    return out.reshape(B, L, D)
